# baseline (speedup 1.0000x reference)
import functools

import jax
import jax.numpy as jnp
from jax import lax
from jax.experimental import pallas as pl
from jax.experimental.pallas import tpu as pltpu
from jax.experimental.pallas import tpu_sc as plsc

D_MODEL = 2048
SEQ_LEN = 4096
BATCH = 4

NC, NS, L = 2, 16, 16
NW = NC * NS
SEQ_PER_W = SEQ_LEN // NW

CHUNK = 4
N_SEQ_CHUNKS = SEQ_PER_W // CHUNK
NXB = 8
ROW_ITERS = 4
ROW_UNROLL = D_MODEL // (ROW_ITERS * L)


def _sc_body(x_hbm, pos_hbm, out_hbm,
             xb0, xb1, xb2, xb3, xb4, xb5, xb6, xb7, pb0, pb1,
             sem_x, sem_p, sem_o):
    c = lax.axis_index("c")
    s = lax.axis_index("s")
    wid = s * NC + c
    seq0 = wid * SEQ_PER_W

    xbufs = (xb0, xb1, xb2, xb3, xb4, xb5, xb6, xb7)
    pbufs = (pb0, pb1)

    def x_row(b, sc):
        return b * SEQ_LEN + seq0 + sc * CHUNK

    def start_x(b, sc, dst):
        pltpu.async_copy(x_hbm.at[pl.ds(x_row(b, sc), CHUNK)], dst, sem_x)

    def start_p(sc, dst):
        pltpu.async_copy(pos_hbm.at[pl.ds(seq0 + sc * CHUNK, CHUNK)], dst,
                         sem_p)

    def start_o(b, sc, src):
        pltpu.async_copy(src, out_hbm.at[pl.ds(x_row(b, sc), CHUNK)], sem_o)

    def wait_x(dst):
        pltpu.make_async_copy(x_hbm.at[pl.ds(0, CHUNK)], dst, sem_x).wait()

    def wait_p(dst):
        pltpu.make_async_copy(pos_hbm.at[pl.ds(0, CHUNK)], dst, sem_p).wait()

    def wait_o(src):
        pltpu.make_async_copy(src, out_hbm.at[pl.ds(0, CHUNK)], sem_o).wait()

    def add_chunk(xbuf, pbuf):
        def body(i, acc):
            base = i * (ROW_UNROLL * L)
            for r in range(CHUNK):
                for j in range(ROW_UNROLL):
                    o = base + j * L
                    plsc.addupdate(xbuf.at[r, pl.ds(o, L)],
                                   pbuf[r, pl.ds(o, L)])
            return acc

        lax.fori_loop(0, ROW_ITERS, body, 0)

    start_p(0, pb0)
    for b in range(BATCH):
        start_x(b, 0, xbufs[b])

    def loop_body(j, acc):
        for u in range(NXB):
            pslot, b = u // BATCH, u % BATCH
            sc = 2 * j + pslot
            xbuf = xbufs[u]
            wait_x(xbuf)
            if u == 0:
                wait_p(pb0)
                start_p(2 * j + 1, pb1)
            if u == BATCH:
                wait_p(pb1)

                @pl.when(j < N_SEQ_CHUNKS // 2 - 1)
                def _():
                    start_p(2 * j + 2, pb0)

            add_chunk(xbuf, pbufs[pslot])
            start_o(b, sc, xbuf)
            if u < BATCH:
                @pl.when(j >= 1)
                def _():
                    wait_o(xbufs[(u + BATCH) % NXB])
                start_x(b, 2 * j + 1, xbufs[(u + BATCH) % NXB])
            else:
                wait_o(xbufs[(u + BATCH) % NXB])

                @pl.when(j < N_SEQ_CHUNKS // 2 - 1)
                def _():
                    start_x(b, 2 * j + 2, xbufs[(u + BATCH) % NXB])
        return acc

    lax.fori_loop(0, N_SEQ_CHUNKS // 2, loop_body, 0)

    for u in range(BATCH, NXB):
        wait_o(xbufs[u])


_sc_add = functools.partial(
    pl.kernel,
    out_type=jax.ShapeDtypeStruct((BATCH * SEQ_LEN, D_MODEL), jnp.float32),
    mesh=plsc.VectorSubcoreMesh(core_axis_name="c", subcore_axis_name="s"),
    scratch_types=[
        pltpu.VMEM((CHUNK, D_MODEL), jnp.float32),
        pltpu.VMEM((CHUNK, D_MODEL), jnp.float32),
        pltpu.VMEM((CHUNK, D_MODEL), jnp.float32),
        pltpu.VMEM((CHUNK, D_MODEL), jnp.float32),
        pltpu.VMEM((CHUNK, D_MODEL), jnp.float32),
        pltpu.VMEM((CHUNK, D_MODEL), jnp.float32),
        pltpu.VMEM((CHUNK, D_MODEL), jnp.float32),
        pltpu.VMEM((CHUNK, D_MODEL), jnp.float32),
        pltpu.VMEM((CHUNK, D_MODEL), jnp.float32),
        pltpu.VMEM((CHUNK, D_MODEL), jnp.float32),
        pltpu.SemaphoreType.DMA,
        pltpu.SemaphoreType.DMA,
        pltpu.SemaphoreType.DMA,
    ],
    compiler_params=pltpu.CompilerParams(use_tc_tiling_on_sc=True),
)(_sc_body)


@jax.jit
def kernel(x, pos_table):
    x2 = x.reshape(BATCH * SEQ_LEN, D_MODEL)
    out = _sc_add(x2, pos_table)
    return out.reshape(x.shape)


# speedup vs baseline: 1.7985x; 1.7985x over previous
import functools

import jax
import jax.numpy as jnp
from jax import lax
from jax.experimental import pallas as pl
from jax.experimental.pallas import tpu as pltpu
from jax.experimental.pallas import tpu_sc as plsc

D_MODEL = 2048
SEQ_LEN = 4096
BATCH = 4

NC, NS, L = 2, 16, 16
NW = NC * NS
SEQ_PER_W = SEQ_LEN // NW

CHUNK = 4
N_SEQ_CHUNKS = SEQ_PER_W // CHUNK
NXB = 8
ROW_ITERS = 16
ROW_UNROLL = D_MODEL // (ROW_ITERS * L)


def _sc_body(x_hbm, pos_hbm, out_hbm,
             xb0, xb1, xb2, xb3, xb4, xb5, xb6, xb7, pb0, pb1,
             sem_x, sem_p, sem_o):
    c = lax.axis_index("c")
    s = lax.axis_index("s")
    wid = s * NC + c
    seq0 = wid * SEQ_PER_W

    xbufs = (xb0, xb1, xb2, xb3, xb4, xb5, xb6, xb7)
    pbufs = (pb0, pb1)

    def x_row(b, sc):
        return b * SEQ_LEN + seq0 + sc * CHUNK

    def start_x(b, sc, dst):
        pltpu.async_copy(x_hbm.at[pl.ds(x_row(b, sc), CHUNK)], dst, sem_x)

    def start_p(sc, dst):
        pltpu.async_copy(pos_hbm.at[pl.ds(seq0 + sc * CHUNK, CHUNK)], dst,
                         sem_p)

    def start_o(b, sc, src):
        pltpu.async_copy(src, out_hbm.at[pl.ds(x_row(b, sc), CHUNK)], sem_o)

    def wait_x(dst):
        pltpu.make_async_copy(x_hbm.at[pl.ds(0, CHUNK)], dst, sem_x).wait()

    def wait_p(dst):
        pltpu.make_async_copy(pos_hbm.at[pl.ds(0, CHUNK)], dst, sem_p).wait()

    def wait_o(src):
        pltpu.make_async_copy(src, out_hbm.at[pl.ds(0, CHUNK)], sem_o).wait()

    def add_chunk(xbuf, pbuf):
        def body(i, acc):
            base = i * (ROW_UNROLL * L)
            for r in range(CHUNK):
                for j in range(ROW_UNROLL):
                    o = base + j * L
                    plsc.addupdate(xbuf.at[r, pl.ds(o, L)],
                                   pbuf[r, pl.ds(o, L)])
            return acc

        lax.fori_loop(0, ROW_ITERS, body, 0)

    start_p(0, pb0)
    for b in range(BATCH):
        start_x(b, 0, xbufs[b])

    def loop_body(j, acc):
        for u in range(NXB):
            pslot, b = u // BATCH, u % BATCH
            sc = 2 * j + pslot
            xbuf = xbufs[u]
            wait_x(xbuf)
            if u == 0:
                wait_p(pb0)
                start_p(2 * j + 1, pb1)
            if u == BATCH:
                wait_p(pb1)

                @pl.when(j < N_SEQ_CHUNKS // 2 - 1)
                def _():
                    start_p(2 * j + 2, pb0)

            add_chunk(xbuf, pbufs[pslot])
            start_o(b, sc, xbuf)
            if u < BATCH:
                @pl.when(j >= 1)
                def _():
                    wait_o(xbufs[(u + BATCH) % NXB])
                start_x(b, 2 * j + 1, xbufs[(u + BATCH) % NXB])
            else:
                wait_o(xbufs[(u + BATCH) % NXB])

                @pl.when(j < N_SEQ_CHUNKS // 2 - 1)
                def _():
                    start_x(b, 2 * j + 2, xbufs[(u + BATCH) % NXB])
        return acc

    lax.fori_loop(0, N_SEQ_CHUNKS // 2, loop_body, 0)

    for u in range(BATCH, NXB):
        wait_o(xbufs[u])


_sc_add = functools.partial(
    pl.kernel,
    out_type=jax.ShapeDtypeStruct((BATCH * SEQ_LEN, D_MODEL), jnp.float32),
    mesh=plsc.VectorSubcoreMesh(core_axis_name="c", subcore_axis_name="s"),
    scratch_types=[
        pltpu.VMEM((CHUNK, D_MODEL), jnp.float32),
        pltpu.VMEM((CHUNK, D_MODEL), jnp.float32),
        pltpu.VMEM((CHUNK, D_MODEL), jnp.float32),
        pltpu.VMEM((CHUNK, D_MODEL), jnp.float32),
        pltpu.VMEM((CHUNK, D_MODEL), jnp.float32),
        pltpu.VMEM((CHUNK, D_MODEL), jnp.float32),
        pltpu.VMEM((CHUNK, D_MODEL), jnp.float32),
        pltpu.VMEM((CHUNK, D_MODEL), jnp.float32),
        pltpu.VMEM((CHUNK, D_MODEL), jnp.float32),
        pltpu.VMEM((CHUNK, D_MODEL), jnp.float32),
        pltpu.SemaphoreType.DMA,
        pltpu.SemaphoreType.DMA,
        pltpu.SemaphoreType.DMA,
    ],
    compiler_params=pltpu.CompilerParams(use_tc_tiling_on_sc=True),
)(_sc_body)


@jax.jit
def kernel(x, pos_table):
    x2 = x.reshape(BATCH * SEQ_LEN, D_MODEL)
    out = _sc_add(x2, pos_table)
    return out.reshape(x.shape)
